# baseline (speedup 1.0000x reference)
import functools

import jax
import jax.numpy as jnp
from jax import lax
from jax.experimental import pallas as pl
from jax.experimental.pallas import tpu as pltpu
from jax.experimental.pallas import tpu_sc as plsc

NC = 2
NS = 16
NW = NC * NS

H_GROUPS = 2
B_GROUPS = 16
CH = 512


def _make_kernel(HIST, BATCH, V, D):
    assert NW == H_GROUPS * B_GROUPS
    assert D == 32 and BATCH % (B_GROUPS * CH) == 0 and CH % 128 == 0
    h_per_w = HIST // H_GROUPS
    b_per_w = BATCH // B_GROUPS
    assert h_per_w * H_GROUPS == HIST
    cpb = b_per_w // CH
    n_chunks = h_per_w * cpb
    assert n_chunks % 2 == 0 and n_chunks >= 6
    TRS = D // 8
    STRIP = (CH // 128) * 1024

    mesh = plsc.VectorSubcoreMesh(core_axis_name="c", subcore_axis_name="s")

    @functools.partial(
        pl.kernel,
        out_type=jax.ShapeDtypeStruct((HIST, TRS, (BATCH // 128) * 1024),
                                      jnp.float32),
        mesh=mesh,
        scratch_types=[
            pltpu.VMEM((CH,), jnp.int32),
            pltpu.VMEM((CH,), jnp.int32),
            pltpu.VMEM((CH, D), jnp.float32),
            pltpu.VMEM((CH, D), jnp.float32),
            pltpu.VMEM((TRS * STRIP,), jnp.float32),
            pltpu.VMEM((TRS * STRIP,), jnp.float32),
            pltpu.SemaphoreType.DMA,
            pltpu.SemaphoreType.DMA,
            pltpu.SemaphoreType.DMA,
            pltpu.SemaphoreType.DMA,
            pltpu.SemaphoreType.DMA,
            pltpu.SemaphoreType.DMA,
        ],
        compiler_params=pltpu.CompilerParams(
            use_tc_tiling_on_sc=False, needs_layout_passes=False),
    )
    def gather_kernel(x_hbm, w_hbm, out_hbm, idx0, idx1, rows0, rows1,
                      tb0, tb1, s_i0, s_i1, s_g0, s_g1, s_w0, s_w1):
        idx_v = [idx0, idx1]
        rows_v = [rows0, rows1]
        tbuf = [tb0, tb1]
        sem_i = [s_i0, s_i1]
        sem_g = [s_g0, s_g1]
        sem_w = [s_w0, s_w1]
        wid = lax.axis_index("s") * NC + lax.axis_index("c")
        hg = wid // B_GROUPS
        bg = wid % B_GROUPS
        h0 = hg * h_per_w
        b0 = bg * b_per_w

        def chunk_hb(c):
            h = h0 + c // cpb
            bb = b0 + (c % cpb) * CH
            return h, bb

        def idx_copy(c, b):
            h, bb = chunk_hb(c)
            return pltpu.make_async_copy(
                x_hbm.at[pl.ds(h * BATCH + bb, CH)], idx_v[b], sem_i[b])

        def gather(b):
            return pltpu.make_async_copy(
                w_hbm.at[idx_v[b]], rows_v[b], sem_g[b])

        def write_descs(c, b):
            h, bb = chunk_hb(c)
            inner0 = (bb // 128) * 1024
            return [
                pltpu.make_async_copy(
                    tbuf[b].at[pl.ds(tr * STRIP, STRIP)],
                    out_hbm.at[h, tr, pl.ds(inner0, STRIP)],
                    sem_w[b])
                for tr in range(TRS)
            ]

        lane = lax.iota(jnp.int32, 16)
        diag = [(lane + s) % 16 + d0 for d0 in range(0, D, 16) for s in range(16)]
        dbase = [(dv // 8) * STRIP + (dv % 8) * 128 + lane for dv in diag]

        def transpose(b):
            rows = rows_v[b]
            tb = tbuf[b]

            @plsc.parallel_loop(0, CH, 16, unroll=2)
            def _(j0):
                rowv = j0 + lane
                scal = (j0 // 128) * 1024 + j0 % 128
                for colv, basev in zip(diag, dbase):
                    plsc.store_scatter(
                        tb, [basev + scal],
                        plsc.load_gather(rows, [rowv, colv]))

        for b in range(2):
            idx_copy(b, b).start()

        def outer(cc, _):
            for b in range(2):
                c = cc * 2 + b
                pb = 1 - b

                idx_copy(c, b).wait()
                gather(b).start()

                @pl.when(c >= 1)
                def _():
                    gather(pb).wait()

                @pl.when(jnp.logical_and(c >= 1, c + 1 < n_chunks))
                def _():
                    idx_copy(c + 1, pb).start()

                @pl.when(c >= 3)
                def _():
                    for d_ in write_descs(c - 3, pb):
                        d_.wait()

                @pl.when(c >= 1)
                def _():
                    transpose(pb)
                    for d_ in write_descs(c - 1, pb):
                        d_.start()
            return 0

        lax.fori_loop(0, n_chunks // 2, outer, 0)

        bl = (n_chunks - 1) % 2
        gather(bl).wait()
        for d_ in write_descs(n_chunks - 3, bl):
            d_.wait()
        transpose(bl)
        for d_ in write_descs(n_chunks - 1, bl):
            d_.start()
        for d_ in write_descs(n_chunks - 2, 1 - bl):
            d_.wait()
        for d_ in write_descs(n_chunks - 1, bl):
            d_.wait()

    return gather_kernel


CHW = 800


def _make_w_transpose(V, D):
    NCH = V // CHW
    assert NCH * CHW == V and CHW % 16 == 0
    NI = -(-NCH // NW)

    mesh = plsc.VectorSubcoreMesh(core_axis_name="c", subcore_axis_name="s")

    @functools.partial(
        pl.kernel,
        out_type=jax.ShapeDtypeStruct((V, D), jnp.float32),
        mesh=mesh,
        scratch_types=[
            pltpu.VMEM((D, CHW), jnp.float32),
            pltpu.VMEM((D, CHW), jnp.float32),
            pltpu.VMEM((CHW, D), jnp.float32),
            pltpu.VMEM((CHW, D), jnp.float32),
            pltpu.SemaphoreType.DMA,
            pltpu.SemaphoreType.DMA,
            pltpu.SemaphoreType.DMA,
            pltpu.SemaphoreType.DMA,
        ],
        compiler_params=pltpu.CompilerParams(
            use_tc_tiling_on_sc=False, needs_layout_passes=False),
    )
    def wt_kernel(wt_hbm, out_hbm, vi0, vi1, vo0, vo1, s_i0, s_i1, s_o0, s_o1):
        vin = [vi0, vi1]
        vout = [vo0, vo1]
        sem_i = [s_i0, s_i1]
        sem_o = [s_o0, s_o1]
        wid = lax.axis_index("s") * NC + lax.axis_index("c")

        def ci_of(i):
            return i * NW + wid

        def in_copy(i, b):
            return pltpu.make_async_copy(
                wt_hbm.at[:, pl.ds(ci_of(i) * CHW, CHW)], vin[b], sem_i[b])

        def out_copy(i, b):
            return pltpu.make_async_copy(
                vout[b], out_hbm.at[pl.ds(ci_of(i) * CHW, CHW), :], sem_o[b])

        lane = lax.iota(jnp.int32, 16)
        diag = [(lane + s) % 16 + d0 for d0 in range(0, D, 16) for s in range(16)]

        def transpose(b):
            src = vin[b]
            dst = vout[b]

            @plsc.parallel_loop(0, CHW, 16, unroll=2)
            def _(v0):
                vvec = v0 + lane
                for colv in diag:
                    plsc.store_scatter(
                        dst, [vvec, colv], plsc.load_gather(src, [colv, vvec]))

        for b in range(2):
            @pl.when(ci_of(b) < NCH)
            def _():
                in_copy(b, b).start()

        def outer(oo, _):
            for b in range(2):
                i = oo * 2 + b
                live = ci_of(i) < NCH

                @pl.when(live)
                def _():
                    in_copy(i, b).wait()

                @pl.when(jnp.logical_and(live, i >= 2))
                def _():
                    out_copy(i - 2, b).wait()

                @pl.when(live)
                def _():
                    transpose(b)
                    out_copy(i, b).start()

                @pl.when(jnp.logical_and(i + 2 < NI, ci_of(i + 2) < NCH))
                def _():
                    in_copy(i + 2, b).start()
            return 0

        lax.fori_loop(0, (NI + 1) // 2, outer, 0)

        for k in (2, 1):
            @pl.when(ci_of(NI - k) < NCH)
            def _():
                out_copy(NI - k, (NI - k) % 2).wait()

    return wt_kernel


def kernel(x, W):
    B_, H = x.shape
    V, D = W.shape
    flat = jnp.transpose(x).reshape(-1).astype(jnp.int32)
    w_rm = _make_w_transpose(V, D)(jnp.transpose(W))
    out = _make_kernel(H, B_, V, D)(flat, w_rm)
    out = out.reshape(H, D // 8, B_ // 128, 8, 128)
    out = jnp.transpose(out, (2, 4, 0, 1, 3))
    return out.reshape(B_, H, D)


# speedup vs baseline: 4.8752x; 4.8752x over previous
import functools

import jax
import jax.numpy as jnp
from jax import lax
from jax.experimental import pallas as pl
from jax.experimental.pallas import tpu as pltpu
from jax.experimental.pallas import tpu_sc as plsc

NC = 2
NS = 16
NW = NC * NS

H_GROUPS = 2
B_GROUPS = 16
CH = 512


def _make_kernel(HIST, BATCH, V, D):
    assert NW == H_GROUPS * B_GROUPS
    assert D == 32 and BATCH % (B_GROUPS * CH) == 0 and CH % 128 == 0
    h_per_w = HIST // H_GROUPS
    b_per_w = BATCH // B_GROUPS
    assert h_per_w * H_GROUPS == HIST
    cpb = b_per_w // CH
    n_chunks = h_per_w * cpb
    assert n_chunks % 2 == 0 and n_chunks >= 6
    TRS = D // 8
    STRIP = (CH // 128) * 1024

    mesh = plsc.VectorSubcoreMesh(core_axis_name="c", subcore_axis_name="s")

    @functools.partial(
        pl.kernel,
        out_type=jax.ShapeDtypeStruct((HIST, TRS, (BATCH // 128) * 1024),
                                      jnp.float32),
        mesh=mesh,
        scratch_types=[
            pltpu.VMEM((CH,), jnp.int32),
            pltpu.VMEM((CH,), jnp.int32),
            pltpu.VMEM((CH, D), jnp.float32),
            pltpu.VMEM((CH, D), jnp.float32),
            pltpu.VMEM((TRS * STRIP,), jnp.float32),
            pltpu.VMEM((TRS * STRIP,), jnp.float32),
            pltpu.SemaphoreType.DMA,
            pltpu.SemaphoreType.DMA,
            pltpu.SemaphoreType.DMA,
            pltpu.SemaphoreType.DMA,
            pltpu.SemaphoreType.DMA,
            pltpu.SemaphoreType.DMA,
        ],
        compiler_params=pltpu.CompilerParams(
            use_tc_tiling_on_sc=False, needs_layout_passes=False),
    )
    def gather_kernel(x_hbm, w_hbm, out_hbm, idx0, idx1, rows0, rows1,
                      tb0, tb1, s_i0, s_i1, s_g0, s_g1, s_w0, s_w1):
        idx_v = [idx0, idx1]
        rows_v = [rows0, rows1]
        tbuf = [tb0, tb1]
        sem_i = [s_i0, s_i1]
        sem_g = [s_g0, s_g1]
        sem_w = [s_w0, s_w1]
        wid = lax.axis_index("s") * NC + lax.axis_index("c")
        hg = wid // B_GROUPS
        bg = wid % B_GROUPS
        h0 = hg * h_per_w
        b0 = bg * b_per_w

        def chunk_hb(c):
            h = h0 + c // cpb
            bb = b0 + (c % cpb) * CH
            return h, bb

        def idx_copy(c, b):
            h, bb = chunk_hb(c)
            return pltpu.make_async_copy(
                x_hbm.at[pl.ds(h * BATCH + bb, CH)], idx_v[b], sem_i[b])

        def gather(b):
            return pltpu.make_async_copy(
                w_hbm.at[idx_v[b]], rows_v[b], sem_g[b])

        def write_descs(c, b):
            h, bb = chunk_hb(c)
            inner0 = (bb // 128) * 1024
            return [
                pltpu.make_async_copy(
                    tbuf[b].at[pl.ds(tr * STRIP, STRIP)],
                    out_hbm.at[h, tr, pl.ds(inner0, STRIP)],
                    sem_w[b])
                for tr in range(TRS)
            ]

        lane = lax.iota(jnp.int32, 16)
        diag = [(lane + s) % 16 + d0 for d0 in range(0, D, 16) for s in range(16)]
        dbase = [(dv // 8) * STRIP + (dv % 8) * 128 + lane for dv in diag]

        def transpose(b):
            rows = rows_v[b]
            tb = tbuf[b]

            @plsc.parallel_loop(0, CH, 16, unroll=2)
            def _(j0):
                rowv = j0 + lane
                scal = (j0 // 128) * 1024 + j0 % 128
                for colv, basev in zip(diag, dbase):
                    plsc.store_scatter(
                        tb, [basev + scal],
                        plsc.load_gather(rows, [rowv, colv]))

        for b in range(2):
            idx_copy(b, b).start()

        def outer(cc, _):
            for b in range(2):
                c = cc * 2 + b
                pb = 1 - b

                idx_copy(c, b).wait()
                gather(b).start()

                @pl.when(c >= 1)
                def _():
                    gather(pb).wait()

                @pl.when(jnp.logical_and(c >= 1, c + 1 < n_chunks))
                def _():
                    idx_copy(c + 1, pb).start()

                @pl.when(c >= 3)
                def _():
                    for d_ in write_descs(c - 3, pb):
                        d_.wait()

                @pl.when(c >= 1)
                def _():
                    transpose(pb)
                    for d_ in write_descs(c - 1, pb):
                        d_.start()
            return 0

        lax.fori_loop(0, n_chunks // 2, outer, 0)

        bl = (n_chunks - 1) % 2
        gather(bl).wait()
        for d_ in write_descs(n_chunks - 3, bl):
            d_.wait()
        transpose(bl)
        for d_ in write_descs(n_chunks - 1, bl):
            d_.start()
        for d_ in write_descs(n_chunks - 2, 1 - bl):
            d_.wait()
        for d_ in write_descs(n_chunks - 1, bl):
            d_.wait()

    return gather_kernel


TCW = 13


def _make_w_transpose(VP, D):
    NTC = VP // 128
    CHW = TCW * 128
    NCH = NTC // TCW
    assert NCH * TCW == NTC and D == 32
    NI = -(-NCH // NW)
    STRIPI = TCW * 1024

    mesh = plsc.VectorSubcoreMesh(core_axis_name="c", subcore_axis_name="s")

    @functools.partial(
        pl.kernel,
        out_type=jax.ShapeDtypeStruct((VP * D,), jnp.float32),
        mesh=mesh,
        scratch_types=[
            pltpu.VMEM((2 * STRIPI,), jnp.float32),
            pltpu.VMEM((2 * STRIPI,), jnp.float32),
            pltpu.VMEM((CHW * D,), jnp.float32),
            pltpu.SemaphoreType.DMA,
            pltpu.SemaphoreType.DMA,
            pltpu.SemaphoreType.DMA,
        ],
        compiler_params=pltpu.CompilerParams(
            use_tc_tiling_on_sc=False, needs_layout_passes=False),
    )
    def wt_kernel(wp_hbm, out_hbm, vi0, vi1, vout, s_i0, s_i1, s_o):
        vin = [vi0, vi1]
        sem_i = [s_i0, s_i1]
        wid = lax.axis_index("s") * NC + lax.axis_index("c")

        def ci_of(i):
            return i * NW + wid

        def in_descs(i, h):
            tc0 = ci_of(i) * TCW
            return [
                pltpu.make_async_copy(
                    wp_hbm.at[pl.ds(((2 * h + t) * NTC + tc0) * 1024, STRIPI)],
                    vin[h].at[pl.ds(t * STRIPI, STRIPI)],
                    sem_i[h])
                for t in range(2)
            ]

        def out_copy(i):
            return pltpu.make_async_copy(
                vout, out_hbm.at[pl.ds(ci_of(i) * CHW * D, CHW * D)], s_o)

        lane = lax.iota(jnp.int32, 16)

        def transpose_half(h):
            src = vin[h]
            dvecs = [(lane + s) % 16 + 16 * h for s in range(16)]
            lbases = [(dv // 8 - 2 * h) * STRIPI + (dv % 8) * 128 + lane
                      for dv in dvecs]
            sbases = [lane * D + dv for dv in dvecs]

            @plsc.parallel_loop(0, CHW, 16, unroll=2)
            def _(v0):
                lscal = (v0 // 128) * 1024 + v0 % 128
                sscal = v0 * D
                for lb, sb in zip(lbases, sbases):
                    plsc.store_scatter(
                        vout, [sb + sscal],
                        plsc.load_gather(src, [lb + lscal]))

        @pl.when(ci_of(0) < NCH)
        def _():
            for d_ in in_descs(0, 0):
                d_.start()

        def body(i, _):
            live = ci_of(i) < NCH

            @pl.when(live)
            def _():
                for d_ in in_descs(i, 0):
                    d_.wait()
                for d_ in in_descs(i, 1):
                    d_.start()

            @pl.when(jnp.logical_and(live, i >= 1))
            def _():
                out_copy(i - 1).wait()

            @pl.when(live)
            def _():
                transpose_half(0)
                for d_ in in_descs(i, 1):
                    d_.wait()

            @pl.when(ci_of(i + 1) < NCH)
            def _():
                for d_ in in_descs(i + 1, 0):
                    d_.start()

            @pl.when(live)
            def _():
                transpose_half(1)
                out_copy(i).start()
            return 0

        lax.fori_loop(0, NI, body, 0)
        out_copy(0).wait()

    return wt_kernel


def kernel(x, W):
    B_, H = x.shape
    V, D = W.shape
    VP = -(-V // 128) * 128
    flat = jnp.transpose(x).reshape(-1).astype(jnp.int32)
    Wp = jnp.pad(W, ((0, VP - V), (0, 0)))
    wp_bytes = jnp.transpose(Wp).reshape(D // 8, 8, VP // 128, 128)
    wp_bytes = jnp.transpose(wp_bytes, (0, 2, 1, 3)).reshape(-1)
    w_rm = _make_w_transpose(VP, D)(wp_bytes).reshape(VP, D)
    out = _make_kernel(H, B_, VP, D)(flat, w_rm)
    out = out.reshape(H, D // 8, B_ // 128, 8, 128)
    out = jnp.transpose(out, (2, 4, 0, 1, 3))
    return out.reshape(B_, H, D)
